# baseline (speedup 1.0000x reference)
import jax
import jax.numpy as jnp
from jax.experimental import pallas as pl
from jax.experimental.pallas import tpu as pltpu

_N = 160000
_BLOCK = 6400
_K = 256
_H1 = 512
_H2 = 64
_G = _N // _BLOCK


def _mlp_kernel(e_ref, w1t_ref, w2t_ref, w3t_ref, o_ref):
    eb = e_ref[...].astype(jnp.bfloat16)
    w1b = w1t_ref[...].astype(jnp.bfloat16)
    h = jax.lax.dot_general(eb, w1b, (((1,), (1,)), ((), ())),
                            preferred_element_type=jnp.float32)
    h = jnp.maximum(h.astype(jnp.bfloat16), jnp.bfloat16(0.0))
    w2b = w2t_ref[...].astype(jnp.bfloat16)
    ht = jax.lax.dot_general(w2b, h, (((1,), (1,)), ((), ())),
                             preferred_element_type=jnp.float32)
    ht = jnp.maximum(ht.astype(jnp.bfloat16), jnp.bfloat16(0.0))
    w3b = w3t_ref[...].astype(jnp.bfloat16)
    row = jax.lax.dot_general(w3b, ht, (((1,), (0,)), ((), ())),
                              preferred_element_type=jnp.float32)
    o_ref[...] = row.reshape(1, 1, _BLOCK)


def kernel(E, w1, w2, w3):
    w1t = jnp.pad(w1.T, ((0, _H1 - w1.shape[1]), (0, 0)))
    w2t = jnp.pad(w2.T, ((0, _H2 - w2.shape[1]), (0, _H1 - w2.shape[0])))
    w3t = jnp.pad(w3.T, ((0, 0), (0, _H2 - w3.shape[0])))
    out3d = pl.pallas_call(
        _mlp_kernel,
        grid=(_G,),
        in_specs=[
            pl.BlockSpec((_BLOCK, _K), lambda i: (i, 0)),
            pl.BlockSpec((_H1, _K), lambda i: (0, 0)),
            pl.BlockSpec((_H2, _H1), lambda i: (0, 0)),
            pl.BlockSpec((1, _H2), lambda i: (0, 0)),
        ],
        out_specs=pl.BlockSpec((1, 1, _BLOCK), lambda i: (i, 0, 0)),
        out_shape=jax.ShapeDtypeStruct((_G, 1, _BLOCK), jnp.float32),
        compiler_params=pltpu.CompilerParams(
            dimension_semantics=("parallel",),
        ),
    )(E, w1t, w2t, w3t)
    return out3d.reshape(_N, 1)


# speedup vs baseline: 1.0899x; 1.0899x over previous
import jax
import jax.numpy as jnp
from jax.experimental import pallas as pl
from jax.experimental.pallas import tpu as pltpu

_N = 160000
_BLOCK = 6400
_K = 256
_H1 = 500
_H2 = 50
_G = _N // _BLOCK
_SUB = _BLOCK // 128


def _mlp_kernel(e_ref, w1t_ref, w2t_ref, w3t_ref, o_ref):
    eb = e_ref[...].astype(jnp.bfloat16)
    w1b = w1t_ref[...].astype(jnp.bfloat16)
    h = jax.lax.dot_general(eb, w1b, (((1,), (1,)), ((), ())),
                            preferred_element_type=jnp.float32)
    h = jnp.maximum(h.astype(jnp.bfloat16), jnp.bfloat16(0.0))
    w2b = w2t_ref[...].astype(jnp.bfloat16)
    h = jax.lax.dot_general(h, w2b, (((1,), (1,)), ((), ())),
                            preferred_element_type=jnp.float32)
    h = jnp.maximum(h.astype(jnp.bfloat16), jnp.bfloat16(0.0))
    h3 = h.reshape(1, _SUB, 128, _H2).astype(jnp.float32)
    w3v = w3t_ref[...].astype(jnp.bfloat16).astype(jnp.float32)
    o_ref[...] = jnp.sum(h3 * w3v.reshape(1, 1, 1, _H2), axis=3)


def kernel(E, w1, w2, w3):
    out3d = pl.pallas_call(
        _mlp_kernel,
        grid=(_G,),
        in_specs=[
            pl.BlockSpec((_BLOCK, _K), lambda i: (i, 0)),
            pl.BlockSpec((_H1, _K), lambda i: (0, 0)),
            pl.BlockSpec((_H2, _H1), lambda i: (0, 0)),
            pl.BlockSpec((1, _H2), lambda i: (0, 0)),
        ],
        out_specs=pl.BlockSpec((1, _SUB, 128), lambda i: (i, 0, 0)),
        out_shape=jax.ShapeDtypeStruct((_G, _SUB, 128), jnp.float32),
        compiler_params=pltpu.CompilerParams(
            dimension_semantics=("parallel",),
        ),
    )(E, w1.T, w2.T, w3.T)
    return out3d.reshape(_N, 1)
